# speedup vs baseline: 1.7262x; 1.7262x over previous
import functools

import jax
import jax.numpy as jnp
from jax import lax
from jax.experimental import pallas as pl
from jax.experimental.pallas import tpu as pltpu
from jax.experimental.pallas import tpu_sc as plsc

_N_CLASSES = 1000
_FEAT = 128
_BATCH = 16384
_NC = 2
_NS = 16
_NW = _NC * _NS
_LANES = 16
_B_PER_W = _BATCH // _NW
_CHUNK = 128
_NCHUNK = _B_PER_W // _CHUNK


def _make_sc_kernel():
    mesh = plsc.VectorSubcoreMesh(core_axis_name="c", subcore_axis_name="s")

    @functools.partial(
        pl.kernel,
        mesh=mesh,
        out_type=jax.ShapeDtypeStruct((_NW, _LANES), jnp.float32),
        scratch_types=[
            pltpu.VMEM((_NCHUNK, _CHUNK), jnp.int32),
            pltpu.VMEM((_CHUNK, _FEAT), jnp.float32),
            pltpu.VMEM((_CHUNK, _FEAT), jnp.float32),
            pltpu.VMEM((_LANES,), jnp.float32),
            pltpu.SemaphoreType.DMA,
        ],
    )
    def sc_kernel(x_hbm, labels_hbm, centers_hbm, out_hbm,
                  idx_v, cbuf, xbuf, acc_v, sem):
        wid = lax.axis_index("s") * _NC + lax.axis_index("c")
        base = wid * _B_PER_W

        pltpu.sync_copy(labels_hbm.at[wid], idx_v)

        acc = jnp.zeros((_LANES,), jnp.float32)
        for j in range(_NCHUNK):
            pltpu.async_copy(centers_hbm.at[idx_v.at[j]], cbuf, sem).wait()
            pltpu.sync_copy(x_hbm.at[pl.ds(base + j * _CHUNK, _CHUNK)], xbuf)

            def row_body(r, a):
                for v in range(_FEAT // _LANES):
                    d = (xbuf[r, pl.ds(v * _LANES, _LANES)]
                         - cbuf[r, pl.ds(v * _LANES, _LANES)])
                    a = a + d * d
                return a

            acc = lax.fori_loop(0, _CHUNK, row_body, acc)

        acc_v[...] = acc
        pltpu.sync_copy(acc_v, out_hbm.at[wid])

    return sc_kernel


_sc_kernel = _make_sc_kernel()


def kernel(x, labels, centers):
    labels_i32 = labels.astype(jnp.int32).reshape(_NW, _NCHUNK, _CHUNK)
    partials = _sc_kernel(x, labels_i32, centers)
    return jnp.sum(partials) / (2.0 * _BATCH)


# speedup vs baseline: 2.0814x; 1.2058x over previous
import functools

import jax
import jax.numpy as jnp
from jax import lax
from jax.experimental import pallas as pl
from jax.experimental.pallas import tpu as pltpu
from jax.experimental.pallas import tpu_sc as plsc

_N_CLASSES = 1000
_FEAT = 128
_BATCH = 16384
_NC = 2
_NS = 16
_NW = _NC * _NS
_LANES = 16
_B_PER_W = _BATCH // _NW
_CHUNK = 128
_NCHUNK = _B_PER_W // _CHUNK


def _make_sc_kernel():
    mesh = plsc.VectorSubcoreMesh(core_axis_name="c", subcore_axis_name="s")

    @functools.partial(
        pl.kernel,
        mesh=mesh,
        out_type=jax.ShapeDtypeStruct((_NW, _LANES), jnp.float32),
        scratch_types=[
            pltpu.VMEM((_NCHUNK, _CHUNK), jnp.int32),
            pltpu.VMEM((2, _CHUNK, _FEAT), jnp.float32),
            pltpu.VMEM((_NCHUNK, _CHUNK, _FEAT), jnp.float32),
            pltpu.VMEM((_LANES,), jnp.float32),
            pltpu.SemaphoreType.DMA,
            pltpu.SemaphoreType.DMA,
        ],
    )
    def sc_kernel(x_hbm, labels_hbm, centers_hbm, out_hbm,
                  idx_v, cbuf, xbuf, acc_v, sem_x, sem_g):
        wid = lax.axis_index("s") * _NC + lax.axis_index("c")
        base = wid * _B_PER_W

        pltpu.sync_copy(labels_hbm.at[wid], idx_v)

        xcps = [
            pltpu.async_copy(
                x_hbm.at[pl.ds(base + j * _CHUNK, _CHUNK)], xbuf.at[j], sem_x)
            for j in range(_NCHUNK)
        ]
        g = pltpu.async_copy(centers_hbm.at[idx_v.at[0]], cbuf.at[0], sem_g)

        acc = jnp.zeros((_LANES,), jnp.float32)
        for j in range(_NCHUNK):
            g.wait()
            if j + 1 < _NCHUNK:
                g = pltpu.async_copy(
                    centers_hbm.at[idx_v.at[j + 1]], cbuf.at[(j + 1) % 2], sem_g)
            xcps[j].wait()

            def row_body(r, a, j=j):
                for v in range(_FEAT // _LANES):
                    d = (xbuf[j, r, pl.ds(v * _LANES, _LANES)]
                         - cbuf[j % 2, r, pl.ds(v * _LANES, _LANES)])
                    a = a + d * d
                return a

            acc = lax.fori_loop(0, _CHUNK, row_body, acc)

        acc_v[...] = acc
        pltpu.sync_copy(acc_v, out_hbm.at[wid])

    return sc_kernel


_sc_kernel = _make_sc_kernel()


def kernel(x, labels, centers):
    labels_i32 = labels.astype(jnp.int32).reshape(_NW, _NCHUNK, _CHUNK)
    partials = _sc_kernel(x, labels_i32, centers)
    return jnp.sum(partials) / (2.0 * _BATCH)


# speedup vs baseline: 2.0973x; 1.0076x over previous
import functools

import jax
import jax.numpy as jnp
from jax import lax
from jax.experimental import pallas as pl
from jax.experimental.pallas import tpu as pltpu
from jax.experimental.pallas import tpu_sc as plsc

_N_CLASSES = 1000
_FEAT = 128
_BATCH = 16384
_NC = 2
_NS = 16
_NW = _NC * _NS
_LANES = 16
_B_PER_W = _BATCH // _NW
_CHUNK = 128
_NCHUNK = _B_PER_W // _CHUNK


def _make_sc_kernel():
    mesh = plsc.VectorSubcoreMesh(core_axis_name="c", subcore_axis_name="s")

    @functools.partial(
        pl.kernel,
        mesh=mesh,
        out_type=jax.ShapeDtypeStruct((_NW, _LANES), jnp.float32),
        scratch_types=[
            pltpu.VMEM((_NCHUNK, _CHUNK), jnp.int32),
            pltpu.VMEM((2, _CHUNK, _FEAT), jnp.float32),
            pltpu.VMEM((_NCHUNK, _CHUNK, _FEAT), jnp.float32),
            pltpu.VMEM((_LANES,), jnp.float32),
            pltpu.SemaphoreType.DMA,
            pltpu.SemaphoreType.DMA,
        ],
    )
    def sc_kernel(x_hbm, labels_hbm, centers_hbm, out_hbm,
                  idx_v, cbuf, xbuf, acc_v, sem_x, sem_g):
        wid = lax.axis_index("s") * _NC + lax.axis_index("c")
        base = wid * _B_PER_W

        pltpu.sync_copy(labels_hbm.at[wid], idx_v)

        xcps = [
            pltpu.async_copy(
                x_hbm.at[pl.ds(base + j * _CHUNK, _CHUNK)], xbuf.at[j], sem_x)
            for j in range(_NCHUNK)
        ]
        g = pltpu.async_copy(centers_hbm.at[idx_v.at[0]], cbuf.at[0], sem_g)

        accs = tuple(jnp.zeros((_LANES,), jnp.float32)
                     for _ in range(_FEAT // _LANES))
        for j in range(_NCHUNK):
            g.wait()
            if j + 1 < _NCHUNK:
                g = pltpu.async_copy(
                    centers_hbm.at[idx_v.at[j + 1]], cbuf.at[(j + 1) % 2], sem_g)
            xcps[j].wait()

            def row_body(r, a, j=j):
                out = []
                for v in range(_FEAT // _LANES):
                    d = (xbuf[j, r, pl.ds(v * _LANES, _LANES)]
                         - cbuf[j % 2, r, pl.ds(v * _LANES, _LANES)])
                    out.append(a[v] + d * d)
                return tuple(out)

            accs = lax.fori_loop(0, _CHUNK, row_body, accs)

        acc = accs[0]
        for v in range(1, _FEAT // _LANES):
            acc = acc + accs[v]
        acc_v[...] = acc
        pltpu.sync_copy(acc_v, out_hbm.at[wid])

    return sc_kernel


_sc_kernel = _make_sc_kernel()


def kernel(x, labels, centers):
    labels_i32 = labels.astype(jnp.int32).reshape(_NW, _NCHUNK, _CHUNK)
    partials = _sc_kernel(x, labels_i32, centers)
    return jnp.sum(partials) / (2.0 * _BATCH)
